# baseline (speedup 1.0000x reference)
import functools

import jax
import jax.numpy as jnp
from jax import lax
from jax.experimental import pallas as pl
from jax.experimental.pallas import tpu as pltpu
from jax.experimental.pallas import tpu_sc as plsc

H = 64
IDX_CHUNK = 128


def _sc_gather(table, idx):
    n = idx.shape[0]
    info = plsc.get_sparse_core_info()
    nw = info.num_cores * info.num_subcores
    assert n % nw == 0
    b_per_w = n // nw
    assert b_per_w % 8 == 0
    sizes = []
    left = b_per_w
    while left > 0:
        s = min(IDX_CHUNK, left)
        sizes.append(s)
        left -= s

    mesh = plsc.VectorSubcoreMesh(core_axis_name="c", subcore_axis_name="s")

    @functools.partial(
        pl.kernel,
        out_type=jax.ShapeDtypeStruct((n, H), jnp.float32),
        mesh=mesh,
        scratch_types=[
            pltpu.VMEM((b_per_w,), jnp.int32),
            pltpu.VMEM((b_per_w, H), jnp.float32),
            pltpu.SemaphoreType.DMA,
        ],
        compiler_params=pltpu.CompilerParams(use_tc_tiling_on_sc=False),
    )
    def k(table_hbm, idx_hbm, out_hbm, idx_v, rows_v, sem):
        wid = lax.axis_index("s") * info.num_cores + lax.axis_index("c")
        base = wid * b_per_w
        pltpu.sync_copy(idx_hbm.at[pl.ds(base, b_per_w)], idx_v)
        copies = []
        off = 0
        for s in sizes:
            copies.append(
                pltpu.async_copy(
                    table_hbm.at[idx_v.at[pl.ds(off, s)]],
                    rows_v.at[pl.ds(off, s)],
                    sem,
                )
            )
            off += s
        for c in copies:
            c.wait()
        pltpu.sync_copy(rows_v, out_hbm.at[pl.ds(base, b_per_w)])

    return k(table, idx)


def _lstm_tc(emb_tm, ctx, h0f, c0f, h0b, c0b, Wf, bf, Wb, bb):
    T, B, _ = emb_tm.shape

    half = T // 2
    assert 2 * half == T

    def body(emb_f, emb_b, h0f_r, c0f_r, h0b_r, c0b_r,
             wf_r, bf_r, wb_r, bb_r, ctx_hbm,
             out_hbm, hf_o, cf_o, hb_o, cb_o,
             ctx_buf, hf_s, cf_s, hb_s, cb_s, hist,
             ctx_sem, out_sem):
        t = pl.program_id(0)
        par = lax.rem(t, 2)
        nxt = lax.rem(t + 1, 2)

        def ctx_copy(d, col, parity):
            return pltpu.make_async_copy(
                ctx_hbm.at[:, col, :], ctx_buf.at[d, parity], ctx_sem.at[d, parity])

        def out_flush(col):
            return pltpu.make_async_copy(
                hist.at[col], out_hbm.at[:, col, :], out_sem)

        @pl.when(t == 0)
        def _():
            ctx_copy(0, 0, 0).start()
            ctx_copy(1, T - 1, 0).start()
            hf_s[0] = h0f_r[:]
            cf_s[:] = c0f_r[:]
            hb_s[0] = h0b_r[:]
            cb_s[:] = c0b_r[:]

        @pl.when(t + 1 < T)
        def _():
            ctx_copy(0, t + 1, nxt).start()
            ctx_copy(1, T - 2 - t, nxt).start()

        @pl.when(t >= half + 2)
        def _():
            out_flush(t - 2).wait()
            out_flush(T + 1 - t).wait()

        ctx_copy(0, t, par).wait()
        ctx_copy(1, T - 1 - t, par).wait()

        def step(emb, ctx_t, h, c, w, b):
            x = jnp.concatenate([emb, ctx_t, h], axis=-1)
            g = jnp.dot(x, w, preferred_element_type=jnp.float32) + b
            s = jax.nn.sigmoid(g)
            i = s[:, 0 * H:1 * H]
            f = s[:, 1 * H:2 * H]
            gg = 2.0 * s[:, 2 * H:3 * H] - 1.0
            o = s[:, 3 * H:4 * H]
            c2 = f * c + i * gg
            h2 = o * jnp.tanh(c2)
            return h2, c2

        hf, cf = step(emb_f[0], ctx_buf[0, par], hf_s[par], cf_s[:],
                      wf_r[:], bf_r[:])
        hf_s[nxt] = hf
        cf_s[:] = cf
        hist[t, :, 0:H] = hf

        hb, cb = step(emb_b[0], ctx_buf[1, par], hb_s[par], cb_s[:],
                      wb_r[:], bb_r[:])
        hb_s[nxt] = hb
        cb_s[:] = cb
        hist[T - 1 - t, :, H:2 * H] = hb

        @pl.when(t >= half)
        def _():
            out_flush(t).start()
            out_flush(T - 1 - t).start()

        @pl.when(t == T - 1)
        def _():
            hf_o[:] = hf
            cf_o[:] = cf
            hb_o[:] = hb
            cb_o[:] = cb
            out_flush(t - 1).wait()
            out_flush(T - t).wait()
            out_flush(t).wait()
            out_flush(T - 1 - t).wait()

    full = lambda shape: pl.BlockSpec(shape, lambda t: (0,) * len(shape))
    tspec = lambda w: pl.BlockSpec((1, B, w), lambda t: (t, 0, 0))
    rspec = lambda w: pl.BlockSpec((1, B, w), lambda t: (T - 1 - t, 0, 0))
    any_spec = pl.BlockSpec(memory_space=pl.ANY)

    in_specs = [
        tspec(H), rspec(H),
        full((B, H)), full((B, H)), full((B, H)), full((B, H)),
        full((4 * H, 4 * H)), full((1, 4 * H)),
        full((4 * H, 4 * H)), full((1, 4 * H)),
        any_spec,
    ]
    out_specs = [
        any_spec,
        full((B, H)), full((B, H)), full((B, H)), full((B, H)),
    ]
    out_shape = [
        jax.ShapeDtypeStruct((B, T, 2 * H), jnp.float32),
        jax.ShapeDtypeStruct((B, H), jnp.float32),
        jax.ShapeDtypeStruct((B, H), jnp.float32),
        jax.ShapeDtypeStruct((B, H), jnp.float32),
        jax.ShapeDtypeStruct((B, H), jnp.float32),
    ]
    scratch = [
        pltpu.VMEM((2, 2, B, 2 * H), jnp.float32),
        pltpu.VMEM((2, B, H), jnp.float32),
        pltpu.VMEM((B, H), jnp.float32),
        pltpu.VMEM((2, B, H), jnp.float32),
        pltpu.VMEM((B, H), jnp.float32),
        pltpu.VMEM((T, B, 2 * H), jnp.float32),
        pltpu.SemaphoreType.DMA((2, 2)),
        pltpu.SemaphoreType.DMA,
    ]
    return pl.pallas_call(
        body,
        grid=(T,),
        in_specs=in_specs,
        out_specs=out_specs,
        out_shape=out_shape,
        scratch_shapes=scratch,
    )(emb_tm, emb_tm, h0f, c0f, h0b, c0b, Wf, bf, Wb, bb, ctx)


def kernel(inputs, context, decoder_hidden_state, decoder_cell_state, table,
           Wih_f, Whh_f, bih_f, bhh_f, Wih_b, Whh_b, bih_b, bhh_b):
    B, T = inputs.shape

    idx_tm = jnp.transpose(inputs).reshape(-1).astype(jnp.int32)
    emb_flat = _sc_gather(table, idx_tm)
    emb_tm = emb_flat.reshape(T, B, H)

    gate_scale = jnp.concatenate(
        [jnp.ones((2 * H,), jnp.float32), jnp.full((H,), 2.0, jnp.float32),
         jnp.ones((H,), jnp.float32)])
    Wf = jnp.concatenate([Wih_f.T, Whh_f.T], axis=0) * gate_scale
    Wb = jnp.concatenate([Wih_b.T, Whh_b.T], axis=0) * gate_scale
    bf = ((bih_f + bhh_f) * gate_scale).reshape(1, -1)
    bb = ((bih_b + bhh_b) * gate_scale).reshape(1, -1)

    out, hf, cf, hb, cb = _lstm_tc(
        emb_tm, context,
        decoder_hidden_state[0], decoder_cell_state[0],
        decoder_hidden_state[1], decoder_cell_state[1],
        Wf, bf, Wb, bb)
    h_n = jnp.stack([hf, hb], axis=0)
    c_n = jnp.stack([cf, cb], axis=0)
    return out, h_n, c_n
